# Initial kernel scaffold; baseline (speedup 1.0000x reference)
import jax
import jax.numpy as jnp
from jax import lax
from jax.experimental import pallas as pl
from jax.experimental.pallas import tpu as pltpu
from jax.experimental.pallas import tpu_sc as plsc

B = 16384
SEQ = 20
NC = 2
NS = 16
NW = NC * NS
BPW = B // NW
TCH = 64
NTCH = BPW // TCH
LCH = 128
NLCH = BPW // LCH
L = 16


def _splat_idx(i):
    return jnp.full((L,), i, dtype=jnp.int32)


def _body(tidx_hbm, ttok_hbm, gtok_hbm, ttab_hbm, xtab_hbm, gtab_hbm,
          out_hbm,
          tidx_v, ttok_v, gtok_v, tbuf, gbuf, trows,
          res_t, res_g, inv_t, n0_t, inv_g, n0_g, ttab0, gtab0, sem):
    wid = lax.axis_index("s") * NC + lax.axis_index("c")
    base = pl.multiple_of(wid * BPW, BPW)

    pltpu.sync_copy(tidx_hbm.at[pl.ds(base, BPW)], tidx_v)
    pltpu.sync_copy(ttok_hbm.at[:, pl.ds(base, BPW)], ttok_v)
    pltpu.sync_copy(gtok_hbm.at[:, pl.ds(base, BPW)], gtok_v)
    pltpu.sync_copy(xtab_hbm.at[0], ttab0)
    pltpu.sync_copy(gtab_hbm.at[0], gtab0)

    for c in range(NLCH):
        pltpu.async_copy(
            ttab_hbm.at[tidx_v.at[pl.ds(c * LCH, LCH)]], trows, sem).wait()
        pltpu.sync_copy(trows,
                        out_hbm.at[pl.ds(base + c * LCH, LCH), pl.ds(0, 32)])

    def cnt_body(i, _):
        off = pl.multiple_of(i * L, L)
        ct = jnp.zeros((L,), jnp.float32)
        cg = jnp.zeros((L,), jnp.float32)
        for p in range(SEQ):
            t = ttok_v[p, pl.ds(off, L)]
            g = gtok_v[p, pl.ds(off, L)]
            ct = ct + jnp.where(t != 0, 1.0, 0.0).astype(jnp.float32)
            cg = cg + jnp.where(g != 0, 1.0, 0.0).astype(jnp.float32)
        inv_t[pl.ds(off, L)] = 1.0 / jnp.maximum(ct, 1.0)
        n0_t[pl.ds(off, L)] = float(SEQ) - ct
        inv_g[pl.ds(off, L)] = 1.0 / jnp.maximum(cg, 1.0)
        n0_g[pl.ds(off, L)] = float(SEQ) - cg
        return 0

    lax.fori_loop(0, BPW // L, cnt_body, 0)

    for c in range(NTCH):
        cps = []
        for p in range(SEQ):
            cps.append(pltpu.async_copy(
                xtab_hbm.at[ttok_v.at[p, pl.ds(c * TCH, TCH)]],
                tbuf.at[p], sem))
        for cp in cps:
            cp.wait()

        def tacc_body(r, _):
            rr = c * TCH + r
            iv = plsc.load_gather(inv_t, [_splat_idx(rr)])
            nv = plsc.load_gather(n0_t, [_splat_idx(rr)])
            for h in range(2):
                s = tbuf[0, r, pl.ds(h * L, L)]
                for p in range(1, SEQ):
                    s = s + tbuf[p, r, pl.ds(h * L, L)]
                t0 = ttab0[pl.ds(h * L, L)]
                res_t[r, pl.ds(h * L, L)] = (s - nv * t0) * iv
            return 0

        lax.fori_loop(0, TCH, tacc_body, 0)
        pltpu.sync_copy(res_t,
                        out_hbm.at[pl.ds(base + c * TCH, TCH), pl.ds(32, 32)])

    for c in range(NTCH):
        cps = []
        for p in range(SEQ):
            cps.append(pltpu.async_copy(
                gtab_hbm.at[gtok_v.at[p, pl.ds(c * TCH, TCH)]],
                gbuf.at[p], sem))
        for cp in cps:
            cp.wait()

        def gacc_body(r, _):
            rr = c * TCH + r
            iv = plsc.load_gather(inv_g, [_splat_idx(rr)])
            nv = plsc.load_gather(n0_g, [_splat_idx(rr)])
            s = gbuf[0, r, :]
            for p in range(1, SEQ):
                s = s + gbuf[p, r, :]
            res_g[r, :] = (s - nv * gtab0[:]) * iv
            return 0

        lax.fori_loop(0, TCH, gacc_body, 0)
        pltpu.sync_copy(res_g,
                        out_hbm.at[pl.ds(base + c * TCH, TCH), pl.ds(64, 16)])


@jax.jit
def _run(tidx, ttok_t, gtok_t, ttab, xtab, gtab):
    mesh = plsc.VectorSubcoreMesh(
        core_axis_name="c", subcore_axis_name="s",
        num_cores=NC, num_subcores=NS)
    return pl.kernel(
        _body,
        out_type=jax.ShapeDtypeStruct((B, 80), jnp.float32),
        mesh=mesh,
        scratch_types=[
            pltpu.VMEM((BPW,), jnp.int32),
            pltpu.VMEM((SEQ, BPW), jnp.int32),
            pltpu.VMEM((SEQ, BPW), jnp.int32),
            pltpu.VMEM((SEQ, TCH, 32), jnp.float32),
            pltpu.VMEM((SEQ, TCH, 16), jnp.float32),
            pltpu.VMEM((LCH, 32), jnp.float32),
            pltpu.VMEM((TCH, 32), jnp.float32),
            pltpu.VMEM((TCH, 16), jnp.float32),
            pltpu.VMEM((BPW,), jnp.float32),
            pltpu.VMEM((BPW,), jnp.float32),
            pltpu.VMEM((BPW,), jnp.float32),
            pltpu.VMEM((BPW,), jnp.float32),
            pltpu.VMEM((32,), jnp.float32),
            pltpu.VMEM((16,), jnp.float32),
            pltpu.SemaphoreType.DMA,
        ],
    )(tidx, ttok_t, gtok_t, ttab, xtab, gtab)


def kernel(movie_title_idx, title_tokens, genre_tokens,
           title_table, text_table, genre_table):
    tidx = movie_title_idx.astype(jnp.int32)
    ttok_t = title_tokens.astype(jnp.int32).T
    gtok_t = genre_tokens.astype(jnp.int32).T
    return _run(tidx, ttok_t, gtok_t, title_table, text_table, genre_table)


# speedup vs baseline: 18.6377x; 18.6377x over previous
import jax
import jax.numpy as jnp
from jax import lax
from jax.experimental import pallas as pl
from jax.experimental.pallas import tpu as pltpu
from jax.experimental.pallas import tpu_sc as plsc

B = 16384
SEQ = 20
NC = 2
NS = 16
NW = NC * NS
BPW = B // NW
TCH = 64
NTCH = BPW // TCH
LCH = 128
NLCH = BPW // LCH
L = 16


def _splat_idx(i):
    return jnp.full((L,), i, dtype=jnp.int32)


def _body(tidx_hbm, ttok_hbm, gtok_hbm, ttab_hbm, xtab_hbm, gtab_hbm,
          out_t_hbm, out_x_hbm, out_g_hbm,
          tidx_v, ttok_v, gtok_v, tbuf, gbuf, trows,
          res_t, res_g, inv_t, n0_t, inv_g, n0_g, ttab0, gtab0, sem):
    wid = lax.axis_index("s") * NC + lax.axis_index("c")
    base = pl.multiple_of(wid * BPW, BPW)

    pltpu.sync_copy(tidx_hbm.at[pl.ds(base, BPW)], tidx_v)
    pltpu.sync_copy(ttok_hbm.at[:, pl.ds(base, BPW)], ttok_v)
    pltpu.sync_copy(gtok_hbm.at[:, pl.ds(base, BPW)], gtok_v)
    pltpu.sync_copy(xtab_hbm.at[0], ttab0)
    pltpu.sync_copy(gtab_hbm.at[0], gtab0)

    for c in range(NLCH):
        pltpu.async_copy(
            ttab_hbm.at[tidx_v.at[pl.ds(c * LCH, LCH)]], trows, sem).wait()
        pltpu.sync_copy(trows, out_t_hbm.at[pl.ds(base + c * LCH, LCH)])

    def cnt_body(i, _):
        off = pl.multiple_of(i * L, L)
        ct = jnp.zeros((L,), jnp.float32)
        cg = jnp.zeros((L,), jnp.float32)
        for p in range(SEQ):
            t = ttok_v[p, pl.ds(off, L)]
            g = gtok_v[p, pl.ds(off, L)]
            ct = ct + jnp.where(t != 0, 1.0, 0.0).astype(jnp.float32)
            cg = cg + jnp.where(g != 0, 1.0, 0.0).astype(jnp.float32)
        inv_t[pl.ds(off, L)] = 1.0 / jnp.maximum(ct, 1.0)
        n0_t[pl.ds(off, L)] = float(SEQ) - ct
        inv_g[pl.ds(off, L)] = 1.0 / jnp.maximum(cg, 1.0)
        n0_g[pl.ds(off, L)] = float(SEQ) - cg
        return 0

    lax.fori_loop(0, BPW // L, cnt_body, 0)

    def text_chunk(c, _):
        off = pl.multiple_of(c * TCH, TCH)
        cps = []
        for p in range(SEQ):
            cps.append(pltpu.async_copy(
                xtab_hbm.at[ttok_v.at[p, pl.ds(off, TCH)]],
                tbuf.at[p], sem))
        for cp in cps:
            cp.wait()

        def grp_body(g, _):
            goff = pl.multiple_of(g * L, L)
            iv_vec = inv_t[pl.ds(pl.multiple_of(off + goff, L), L)]
            nv_vec = n0_t[pl.ds(pl.multiple_of(off + goff, L), L)]
            for j in range(L):
                r = goff + j
                iv = iv_vec[j]
                nv = nv_vec[j]
                for h in range(2):
                    s = tbuf[0, r, pl.ds(h * L, L)]
                    for p in range(1, SEQ):
                        s = s + tbuf[p, r, pl.ds(h * L, L)]
                    t0 = ttab0[pl.ds(h * L, L)]
                    res_t[r, pl.ds(h * L, L)] = (s - nv * t0) * iv
            return 0

        lax.fori_loop(0, TCH // L, grp_body, 0)
        pltpu.sync_copy(res_t, out_x_hbm.at[pl.ds(base + off, TCH)])
        return 0

    lax.fori_loop(0, NTCH, text_chunk, 0)

    def genre_chunk(c, _):
        off = pl.multiple_of(c * TCH, TCH)
        cps = []
        for p in range(SEQ):
            cps.append(pltpu.async_copy(
                gtab_hbm.at[gtok_v.at[p, pl.ds(off, TCH)]],
                gbuf.at[p], sem))
        for cp in cps:
            cp.wait()

        def grp_body(g, _):
            goff = pl.multiple_of(g * L, L)
            iv_vec = inv_g[pl.ds(pl.multiple_of(off + goff, L), L)]
            nv_vec = n0_g[pl.ds(pl.multiple_of(off + goff, L), L)]
            for j in range(L):
                r = goff + j
                iv = iv_vec[j]
                nv = nv_vec[j]
                s = gbuf[0, r, :]
                for p in range(1, SEQ):
                    s = s + gbuf[p, r, :]
                res_g[r, :] = (s - nv * gtab0[:]) * iv
            return 0

        lax.fori_loop(0, TCH // L, grp_body, 0)
        pltpu.sync_copy(res_g, out_g_hbm.at[pl.ds(base + off, TCH)])
        return 0

    lax.fori_loop(0, NTCH, genre_chunk, 0)


@jax.jit
def _run(tidx, ttok_t, gtok_t, ttab, xtab, gtab):
    mesh = plsc.VectorSubcoreMesh(
        core_axis_name="c", subcore_axis_name="s",
        num_cores=NC, num_subcores=NS)
    return pl.kernel(
        _body,
        out_type=[
            jax.ShapeDtypeStruct((B, 32), jnp.float32),
            jax.ShapeDtypeStruct((B, 32), jnp.float32),
            jax.ShapeDtypeStruct((B, 16), jnp.float32),
        ],
        mesh=mesh,
        scratch_types=[
            pltpu.VMEM((BPW,), jnp.int32),
            pltpu.VMEM((SEQ, BPW), jnp.int32),
            pltpu.VMEM((SEQ, BPW), jnp.int32),
            pltpu.VMEM((SEQ, TCH, 32), jnp.float32),
            pltpu.VMEM((SEQ, TCH, 16), jnp.float32),
            pltpu.VMEM((LCH, 32), jnp.float32),
            pltpu.VMEM((TCH, 32), jnp.float32),
            pltpu.VMEM((TCH, 16), jnp.float32),
            pltpu.VMEM((BPW,), jnp.float32),
            pltpu.VMEM((BPW,), jnp.float32),
            pltpu.VMEM((BPW,), jnp.float32),
            pltpu.VMEM((BPW,), jnp.float32),
            pltpu.VMEM((32,), jnp.float32),
            pltpu.VMEM((16,), jnp.float32),
            pltpu.SemaphoreType.DMA,
        ],
        compiler_params=pltpu.CompilerParams(use_tc_tiling_on_sc=False),
    )(tidx, ttok_t, gtok_t, ttab, xtab, gtab)


def kernel(movie_title_idx, title_tokens, genre_tokens,
           title_table, text_table, genre_table):
    tidx = movie_title_idx.astype(jnp.int32)
    ttok_t = title_tokens.astype(jnp.int32).T
    gtok_t = genre_tokens.astype(jnp.int32).T
    out_t, out_x, out_g = _run(tidx, ttok_t, gtok_t,
                               title_table, text_table, genre_table)
    return jnp.concatenate([out_t, out_x, out_g], axis=1)


# speedup vs baseline: 20.0474x; 1.0756x over previous
import jax
import jax.numpy as jnp
from jax import lax
from jax.experimental import pallas as pl
from jax.experimental.pallas import tpu as pltpu
from jax.experimental.pallas import tpu_sc as plsc

B = 16384
SEQ = 20
NC = 2
NS = 16
NW = NC * NS
BPW = B // NW
TCH = 32
NTCH = BPW // TCH
LCH = 128
NLCH = BPW // LCH
L = 16


def _splat_idx(i):
    return jnp.full((L,), i, dtype=jnp.int32)


def _body(tidx_hbm, ttok_hbm, gtok_hbm, ttab_hbm, xtab_hbm, gtab_hbm,
          out_t_hbm, out_x_hbm, out_g_hbm,
          tidx_v, ttok_v, gtok_v, tbuf, gbuf, trows,
          res_t, res_g, inv_t, n0_t, inv_g, n0_g, ttab0, gtab0,
          sem, gsem, hsem, wsem, vsem):
    wid = lax.axis_index("s") * NC + lax.axis_index("c")
    base = pl.multiple_of(wid * BPW, BPW)

    pltpu.sync_copy(tidx_hbm.at[pl.ds(base, BPW)], tidx_v)
    pltpu.sync_copy(ttok_hbm.at[:, pl.ds(base, BPW)], ttok_v)
    pltpu.sync_copy(gtok_hbm.at[:, pl.ds(base, BPW)], gtok_v)
    pltpu.sync_copy(xtab_hbm.at[0], ttab0)
    pltpu.sync_copy(gtab_hbm.at[0], gtab0)

    for c in range(NLCH):
        pltpu.async_copy(
            ttab_hbm.at[tidx_v.at[pl.ds(c * LCH, LCH)]], trows, sem).wait()
        pltpu.sync_copy(trows, out_t_hbm.at[pl.ds(base + c * LCH, LCH)])

    def cnt_body(i, _):
        off = pl.multiple_of(i * L, L)
        ct = jnp.zeros((L,), jnp.float32)
        cg = jnp.zeros((L,), jnp.float32)
        for p in range(SEQ):
            t = ttok_v[p, pl.ds(off, L)]
            g = gtok_v[p, pl.ds(off, L)]
            ct = ct + jnp.where(t != 0, 1.0, 0.0).astype(jnp.float32)
            cg = cg + jnp.where(g != 0, 1.0, 0.0).astype(jnp.float32)
        inv_t[pl.ds(off, L)] = 1.0 / jnp.maximum(ct, 1.0)
        n0_t[pl.ds(off, L)] = float(SEQ) - ct
        inv_g[pl.ds(off, L)] = 1.0 / jnp.maximum(cg, 1.0)
        n0_g[pl.ds(off, L)] = float(SEQ) - cg
        return 0

    lax.fori_loop(0, BPW // L, cnt_body, 0)

    def fire_text(c, b):
        off = pl.multiple_of(c * TCH, TCH)
        for p in range(SEQ):
            pltpu.async_copy(
                xtab_hbm.at[ttok_v.at[p, pl.ds(off, TCH)]],
                tbuf.at[b, p], gsem.at[b])

    def drain_text(c, b):
        for p in range(SEQ):
            pltpu.make_async_copy(
                xtab_hbm.at[ttok_v.at[p, pl.ds(0, TCH)]],
                tbuf.at[b, p], gsem.at[b]).wait()

    def text_compute(c, b):
        off = pl.multiple_of(c * TCH, TCH)

        def grp_body(g, _):
            goff = pl.multiple_of(g * L, L)
            iv_vec = inv_t[pl.ds(pl.multiple_of(off + goff, L), L)]
            nv_vec = n0_t[pl.ds(pl.multiple_of(off + goff, L), L)]
            for j in range(L):
                r = goff + j
                iv = iv_vec[j]
                nv = nv_vec[j]
                for h in range(2):
                    s = tbuf[b, 0, r, pl.ds(h * L, L)]
                    for p in range(1, SEQ):
                        s = s + tbuf[b, p, r, pl.ds(h * L, L)]
                    t0 = ttab0[pl.ds(h * L, L)]
                    res_t[b, r, pl.ds(h * L, L)] = (s - nv * t0) * iv
            return 0

        lax.fori_loop(0, TCH // L, grp_body, 0)
        pltpu.async_copy(res_t.at[b], out_x_hbm.at[pl.ds(base + off, TCH)],
                         wsem.at[b])

    def drain_res_t(b):
        pltpu.make_async_copy(res_t.at[b], out_x_hbm.at[pl.ds(0, TCH)],
                              wsem.at[b]).wait()

    def fire_genre(c, b):
        off = pl.multiple_of(c * TCH, TCH)
        for p in range(SEQ):
            pltpu.async_copy(
                gtab_hbm.at[gtok_v.at[p, pl.ds(off, TCH)]],
                gbuf.at[b, p], hsem.at[b])

    def drain_genre(c, b):
        for p in range(SEQ):
            pltpu.make_async_copy(
                gtab_hbm.at[gtok_v.at[p, pl.ds(0, TCH)]],
                gbuf.at[b, p], hsem.at[b]).wait()

    def genre_compute(c, b):
        off = pl.multiple_of(c * TCH, TCH)

        def grp_body(g, _):
            goff = pl.multiple_of(g * L, L)
            iv_vec = inv_g[pl.ds(pl.multiple_of(off + goff, L), L)]
            nv_vec = n0_g[pl.ds(pl.multiple_of(off + goff, L), L)]
            for j in range(L):
                r = goff + j
                iv = iv_vec[j]
                nv = nv_vec[j]
                s = gbuf[b, 0, r, :]
                for p in range(1, SEQ):
                    s = s + gbuf[b, p, r, :]
                res_g[b, r, :] = (s - nv * gtab0[:]) * iv
            return 0

        lax.fori_loop(0, TCH // L, grp_body, 0)
        pltpu.async_copy(res_g.at[b], out_g_hbm.at[pl.ds(base + off, TCH)],
                         vsem.at[b])

    def drain_res_g(b):
        pltpu.make_async_copy(res_g.at[b], out_g_hbm.at[pl.ds(0, TCH)],
                              vsem.at[b]).wait()

    fire_text(0, 0)
    fire_text(1, 1)

    def text_pair(c0, _):
        for b in range(2):
            c = c0 + b
            drain_text(c, b)

            @pl.when(c >= 2)
            def _():
                drain_res_t(b)

            text_compute(c, b)

            @pl.when(c + 2 < NTCH)
            def _():
                fire_text(c + 2, b)
        return 0

    lax.fori_loop(0, NTCH // 2, lambda i, _: text_pair(i * 2, _), 0)
    drain_res_t(0)
    drain_res_t(1)

    fire_genre(0, 0)
    fire_genre(1, 1)

    def genre_pair(c0, _):
        for b in range(2):
            c = c0 + b
            drain_genre(c, b)

            @pl.when(c >= 2)
            def _():
                drain_res_g(b)

            genre_compute(c, b)

            @pl.when(c + 2 < NTCH)
            def _():
                fire_genre(c + 2, b)
        return 0

    lax.fori_loop(0, NTCH // 2, lambda i, _: genre_pair(i * 2, _), 0)
    drain_res_g(0)
    drain_res_g(1)


@jax.jit
def _run(tidx, ttok_t, gtok_t, ttab, xtab, gtab):
    mesh = plsc.VectorSubcoreMesh(
        core_axis_name="c", subcore_axis_name="s",
        num_cores=NC, num_subcores=NS)
    return pl.kernel(
        _body,
        out_type=[
            jax.ShapeDtypeStruct((B, 32), jnp.float32),
            jax.ShapeDtypeStruct((B, 32), jnp.float32),
            jax.ShapeDtypeStruct((B, 16), jnp.float32),
        ],
        mesh=mesh,
        scratch_types=[
            pltpu.VMEM((BPW,), jnp.int32),
            pltpu.VMEM((SEQ, BPW), jnp.int32),
            pltpu.VMEM((SEQ, BPW), jnp.int32),
            pltpu.VMEM((2, SEQ, TCH, 32), jnp.float32),
            pltpu.VMEM((2, SEQ, TCH, 16), jnp.float32),
            pltpu.VMEM((LCH, 32), jnp.float32),
            pltpu.VMEM((2, TCH, 32), jnp.float32),
            pltpu.VMEM((2, TCH, 16), jnp.float32),
            pltpu.VMEM((BPW,), jnp.float32),
            pltpu.VMEM((BPW,), jnp.float32),
            pltpu.VMEM((BPW,), jnp.float32),
            pltpu.VMEM((BPW,), jnp.float32),
            pltpu.VMEM((32,), jnp.float32),
            pltpu.VMEM((16,), jnp.float32),
            pltpu.SemaphoreType.DMA,
            pltpu.SemaphoreType.DMA((2,)),
            pltpu.SemaphoreType.DMA((2,)),
            pltpu.SemaphoreType.DMA((2,)),
            pltpu.SemaphoreType.DMA((2,)),
        ],
        compiler_params=pltpu.CompilerParams(use_tc_tiling_on_sc=False),
    )(tidx, ttok_t, gtok_t, ttab, xtab, gtab)


def kernel(movie_title_idx, title_tokens, genre_tokens,
           title_table, text_table, genre_table):
    tidx = movie_title_idx.astype(jnp.int32)
    ttok_t = title_tokens.astype(jnp.int32).T
    gtok_t = genre_tokens.astype(jnp.int32).T
    out_t, out_x, out_g = _run(tidx, ttok_t, gtok_t,
                               title_table, text_table, genre_table)
    return jnp.concatenate([out_t, out_x, out_g], axis=1)
